# Initial kernel scaffold; baseline (speedup 1.0000x reference)
import jax
import jax.numpy as jnp
from jax.experimental import pallas as pl


def kernel(x, edge_index, W1, b1, W2, b2):
    raise NotImplementedError("write your pallas kernel here")


# speedup vs baseline: 8.2998x; 8.2998x over previous
import functools

import jax
import jax.numpy as jnp
from jax import lax
from jax.experimental import pallas as pl
from jax.experimental.pallas import tpu as pltpu
from jax.experimental.pallas import tpu_sc as plsc

N = 10000
E = 320000
F_IN = 128
F_HID = 128
F_OUT = 64

K = 128
NW = 32
CH = 80
TOT = NW * K * CH
NP = 10240
NPAD = NP + 16
SLAB = NP // 16

_mesh = plsc.VectorSubcoreMesh(core_axis_name="c", subcore_axis_name="s")


def _deg_body(rows_hbm, ones_hbm, zeros_hbm, out_hbm, idx_s, ones_v, hist_sh):
    c = lax.axis_index("c")
    s = lax.axis_index("s")
    wid = c * 16 + s
    pltpu.sync_copy(rows_hbm.at[pl.ds(wid * CH, CH)], idx_s)
    pltpu.sync_copy(ones_hbm, ones_v)
    pltpu.sync_copy(zeros_hbm, hist_sh.at[pl.ds(s * SLAB, SLAB)])
    plsc.subcore_barrier()

    def step(j, carry):
        pltpu.sync_copy(ones_v, hist_sh.at[idx_s.at[j]], add=True)
        return carry

    lax.fori_loop(0, CH, step, 0)
    plsc.subcore_barrier()
    pltpu.sync_copy(hist_sh.at[pl.ds(s * SLAB, SLAB)],
                    out_hbm.at[c].at[pl.ds(s * SLAB, SLAB)])


_deg_kernel = pl.kernel(
    _deg_body,
    out_type=jax.ShapeDtypeStruct((2, NP, 128), jnp.float32),
    mesh=_mesh,
    scratch_types=[
        pltpu.VMEM((CH, K), jnp.int32),
        pltpu.VMEM((K, 128), jnp.float32),
        pltpu.VMEM_SHARED((NPAD, 128), jnp.float32),
    ],
)


def _msg_body(h_hbm, rowg_hbm, cols_hbm, z_hbm, out_hbm, idx_g, idx_s, rows_v,
              acc_sh, sem):
    c = lax.axis_index("c")
    s = lax.axis_index("s")
    wid = c * 16 + s
    pltpu.sync_copy(rowg_hbm.at[pl.ds(wid * CH, CH)], idx_g)
    pltpu.sync_copy(cols_hbm.at[pl.ds(wid * CH, CH)], idx_s)

    @pl.when(c == 0)
    def _():
        pltpu.sync_copy(h_hbm.at[pl.ds(s * SLAB, SLAB)],
                        acc_sh.at[pl.ds(s * SLAB, SLAB)])

    @pl.when(c == 1)
    def _():
        pltpu.sync_copy(z_hbm, acc_sh.at[pl.ds(s * SLAB, SLAB)])

    plsc.subcore_barrier()

    def step(j, carry):
        pltpu.async_copy(h_hbm.at[idx_g.at[j]], rows_v, sem).wait()
        pltpu.sync_copy(rows_v, acc_sh.at[idx_s.at[j]], add=True)
        return carry

    lax.fori_loop(0, CH, step, 0)
    plsc.subcore_barrier()
    pltpu.sync_copy(acc_sh.at[pl.ds(s * SLAB, SLAB)],
                    out_hbm.at[c].at[pl.ds(s * SLAB, SLAB)])


def _make_msg_kernel(F):
    return pl.kernel(
        _msg_body,
        out_type=jax.ShapeDtypeStruct((2, NP, F), jnp.float32),
        mesh=_mesh,
        scratch_types=[
            pltpu.VMEM((CH, K), jnp.int32),
            pltpu.VMEM((CH, K), jnp.int32),
            pltpu.VMEM((K, F), jnp.float32),
            pltpu.VMEM_SHARED((NPAD, F), jnp.float32),
            pltpu.SemaphoreType.DMA,
        ],
    )


_msg_hid = _make_msg_kernel(F_HID)
_msg_out = _make_msg_kernel(F_HID)

_B = 640


def _dis_of(dp_ref):
    deg = dp_ref[0, :, 0] + dp_ref[1, :, 0] + 1.0
    return lax.rsqrt(deg)


def _mm1_body(x_ref, w_ref, b_ref, dp_ref, o_ref):
    dis = _dis_of(dp_ref)
    h = jnp.dot(x_ref[...], w_ref[...], preferred_element_type=jnp.float32)
    o_ref[...] = (h + b_ref[...]) * dis[:, None]


_mm1 = pl.pallas_call(
    _mm1_body,
    grid=(NP // _B,),
    in_specs=[
        pl.BlockSpec((_B, F_IN), lambda i: (i, 0)),
        pl.BlockSpec((F_IN, F_HID), lambda i: (0, 0)),
        pl.BlockSpec((1, F_HID), lambda i: (0, 0)),
        pl.BlockSpec((2, _B, 128), lambda i: (0, i, 0)),
    ],
    out_specs=pl.BlockSpec((_B, F_HID), lambda i: (i, 0)),
    out_shape=jax.ShapeDtypeStruct((NP, F_HID), jnp.float32),
)


def _post1_body(a_ref, dp_ref, w_ref, b_ref, o_ref):
    dis = _dis_of(dp_ref)
    z = jnp.maximum((a_ref[0] + a_ref[1]) * dis[:, None], 0.0)
    h = jnp.dot(z, w_ref[...], preferred_element_type=jnp.float32)
    o_ref[...] = (h + b_ref[...]) * dis[:, None]


_post1 = pl.pallas_call(
    _post1_body,
    grid=(NP // _B,),
    in_specs=[
        pl.BlockSpec((2, _B, F_HID), lambda i: (0, i, 0)),
        pl.BlockSpec((2, _B, 128), lambda i: (0, i, 0)),
        pl.BlockSpec((F_HID, F_HID), lambda i: (0, 0)),
        pl.BlockSpec((1, F_HID), lambda i: (0, 0)),
    ],
    out_specs=pl.BlockSpec((_B, F_HID), lambda i: (i, 0)),
    out_shape=jax.ShapeDtypeStruct((NP, F_HID), jnp.float32),
)


def _post2_body(a_ref, dp_ref, o_ref):
    dis = _dis_of(dp_ref)
    z = (a_ref[0, :, :F_OUT] + a_ref[1, :, :F_OUT]) * dis[:, None]
    m = jnp.max(z, axis=1, keepdims=True)
    e = jnp.exp(z - m)
    o_ref[...] = (z - m) - jnp.log(jnp.sum(e, axis=1, keepdims=True))


_post2 = pl.pallas_call(
    _post2_body,
    grid=(NP // _B,),
    in_specs=[
        pl.BlockSpec((2, _B, F_HID), lambda i: (0, i, 0)),
        pl.BlockSpec((2, _B, 128), lambda i: (0, i, 0)),
    ],
    out_specs=pl.BlockSpec((_B, F_OUT), lambda i: (i, 0)),
    out_shape=jax.ShapeDtypeStruct((NP, F_OUT), jnp.float32),
)


def kernel(x, edge_index, W1, b1, W2, b2):
    row = edge_index[0]
    col = edge_index[1]
    pad0 = jnp.zeros((TOT - E,), jnp.int32)
    padn = jnp.full((TOT - E,), NP, jnp.int32)
    row_g = jnp.concatenate([row, pad0]).reshape(TOT // K, K)
    row_s = jnp.concatenate([row, padn]).reshape(TOT // K, K)
    col_s = jnp.concatenate([col, padn]).reshape(TOT // K, K)
    ones128 = jnp.ones((K, 128), jnp.float32)
    zeros128 = jnp.zeros((SLAB, 128), jnp.float32)
    x_p = jnp.pad(x, ((0, NP - N), (0, 0)))

    W2Tp = jnp.pad(W2.T, ((0, 0), (0, F_HID - F_OUT)))
    b2p = jnp.pad(b2, (0, F_HID - F_OUT))[None, :]

    deg_parts = _deg_kernel(row_s, ones128, zeros128)
    h1 = _mm1(x_p, W1.T, b1[None, :], deg_parts)
    acc1 = _msg_hid(h1, row_g, col_s, zeros128)
    h2 = _post1(acc1, deg_parts, W2Tp, b2p)
    acc2 = _msg_out(h2, row_g, col_s, zeros128)
    return _post2(acc2, deg_parts)[:N]


# speedup vs baseline: 9.1920x; 1.1075x over previous
import functools

import jax
import jax.numpy as jnp
from jax import lax
from jax.experimental import pallas as pl
from jax.experimental.pallas import tpu as pltpu
from jax.experimental.pallas import tpu_sc as plsc

N = 10000
E = 320000
F_IN = 128
F_HID = 128
F_OUT = 64

K = 128
NW = 32
CH = 80
TOT = NW * K * CH
NP = 10240
NPAD = NP + 16
SLAB = NP // 16

_mesh = plsc.VectorSubcoreMesh(core_axis_name="c", subcore_axis_name="s")


def _deg_body(rows_hbm, ones_hbm, zeros_hbm, out_hbm, idx_s, ones_v, hist_sh):
    c = lax.axis_index("c")
    s = lax.axis_index("s")
    wid = c * 16 + s
    pltpu.sync_copy(rows_hbm.at[pl.ds(wid * CH, CH)], idx_s)
    pltpu.sync_copy(ones_hbm, ones_v)
    pltpu.sync_copy(zeros_hbm, hist_sh.at[pl.ds(s * SLAB, SLAB)])
    plsc.subcore_barrier()

    def step(j, carry):
        pltpu.sync_copy(ones_v, hist_sh.at[idx_s.at[j]], add=True)
        return carry

    lax.fori_loop(0, CH, step, 0)
    plsc.subcore_barrier()
    pltpu.sync_copy(hist_sh.at[pl.ds(s * SLAB, SLAB)],
                    out_hbm.at[c].at[pl.ds(s * SLAB, SLAB)])


_deg_kernel = pl.kernel(
    _deg_body,
    out_type=jax.ShapeDtypeStruct((2, NP, 128), jnp.float32),
    mesh=_mesh,
    scratch_types=[
        pltpu.VMEM((CH, K), jnp.int32),
        pltpu.VMEM((K, 128), jnp.float32),
        pltpu.VMEM_SHARED((NPAD, 128), jnp.float32),
    ],
)


_NBUF = 2
_CHP = 40


def _msg_body(h_hbm, rowg_hbm, cols_hbm, z_hbm, out_hbm, idx_g, idx_s,
              b0, b1, s0, s1, acc_sh):
    bufs = (b0, b1)
    sems = (s0, s1)
    c = lax.axis_index("c")
    s = lax.axis_index("s")
    wid = c * 16 + s

    @pl.when(c == 0)
    def _():
        pltpu.sync_copy(h_hbm.at[pl.ds(s * SLAB, SLAB)],
                        acc_sh.at[pl.ds(s * SLAB, SLAB)])

    @pl.when(c == 1)
    def _():
        pltpu.sync_copy(z_hbm, acc_sh.at[pl.ds(s * SLAB, SLAB)])

    plsc.subcore_barrier()

    for p in range(CH // _CHP):
        base = wid * CH + p * _CHP
        pltpu.sync_copy(rowg_hbm.at[pl.ds(base, _CHP)], idx_g)
        pltpu.sync_copy(cols_hbm.at[pl.ds(base, _CHP)], idx_s)
        for b in range(_NBUF):
            pltpu.async_copy(h_hbm.at[idx_g.at[b]], bufs[b], sems[b])

        def round_(t, carry):
            j = t * _NBUF
            for b in range(_NBUF):
                jj = j + b
                pltpu.make_async_copy(h_hbm.at[idx_g.at[jj]], bufs[b],
                                      sems[b]).wait()
                pltpu.sync_copy(bufs[b], acc_sh.at[idx_s.at[jj]], add=True)

                @pl.when(jj + _NBUF < _CHP)
                def _():
                    pltpu.async_copy(h_hbm.at[idx_g.at[jj + _NBUF]], bufs[b],
                                     sems[b])
            return carry

        lax.fori_loop(0, _CHP // _NBUF, round_, 0)

    plsc.subcore_barrier()
    pltpu.sync_copy(acc_sh.at[pl.ds(s * SLAB, SLAB)],
                    out_hbm.at[c].at[pl.ds(s * SLAB, SLAB)])


def _make_msg_kernel(F):
    return pl.kernel(
        _msg_body,
        out_type=jax.ShapeDtypeStruct((2, NP, F), jnp.float32),
        mesh=_mesh,
        scratch_types=(
            [pltpu.VMEM((_CHP, K), jnp.int32),
             pltpu.VMEM((_CHP, K), jnp.int32)]
            + [pltpu.VMEM((K, F), jnp.float32)] * _NBUF
            + [pltpu.SemaphoreType.DMA] * _NBUF
            + [pltpu.VMEM_SHARED((NPAD, F), jnp.float32)]
        ),
    )


_msg_hid = _make_msg_kernel(F_HID)
_msg_out = _make_msg_kernel(F_HID)

_B = 640


def _dis_of(dp_ref):
    deg = dp_ref[0, :, 0] + dp_ref[1, :, 0] + 1.0
    return lax.rsqrt(deg)


def _mm1_body(x_ref, w_ref, b_ref, dp_ref, o_ref):
    dis = _dis_of(dp_ref)
    h = jnp.dot(x_ref[...], w_ref[...], preferred_element_type=jnp.float32)
    o_ref[...] = (h + b_ref[...]) * dis[:, None]


_mm1 = pl.pallas_call(
    _mm1_body,
    grid=(NP // _B,),
    in_specs=[
        pl.BlockSpec((_B, F_IN), lambda i: (i, 0)),
        pl.BlockSpec((F_IN, F_HID), lambda i: (0, 0)),
        pl.BlockSpec((1, F_HID), lambda i: (0, 0)),
        pl.BlockSpec((2, _B, 128), lambda i: (0, i, 0)),
    ],
    out_specs=pl.BlockSpec((_B, F_HID), lambda i: (i, 0)),
    out_shape=jax.ShapeDtypeStruct((NP, F_HID), jnp.float32),
)


def _post1_body(a_ref, dp_ref, w_ref, b_ref, o_ref):
    dis = _dis_of(dp_ref)
    z = jnp.maximum((a_ref[0] + a_ref[1]) * dis[:, None], 0.0)
    h = jnp.dot(z, w_ref[...], preferred_element_type=jnp.float32)
    o_ref[...] = (h + b_ref[...]) * dis[:, None]


_post1 = pl.pallas_call(
    _post1_body,
    grid=(NP // _B,),
    in_specs=[
        pl.BlockSpec((2, _B, F_HID), lambda i: (0, i, 0)),
        pl.BlockSpec((2, _B, 128), lambda i: (0, i, 0)),
        pl.BlockSpec((F_HID, F_HID), lambda i: (0, 0)),
        pl.BlockSpec((1, F_HID), lambda i: (0, 0)),
    ],
    out_specs=pl.BlockSpec((_B, F_HID), lambda i: (i, 0)),
    out_shape=jax.ShapeDtypeStruct((NP, F_HID), jnp.float32),
)


def _post2_body(a_ref, dp_ref, o_ref):
    dis = _dis_of(dp_ref)
    z = (a_ref[0, :, :F_OUT] + a_ref[1, :, :F_OUT]) * dis[:, None]
    m = jnp.max(z, axis=1, keepdims=True)
    e = jnp.exp(z - m)
    o_ref[...] = (z - m) - jnp.log(jnp.sum(e, axis=1, keepdims=True))


_post2 = pl.pallas_call(
    _post2_body,
    grid=(NP // _B,),
    in_specs=[
        pl.BlockSpec((2, _B, F_HID), lambda i: (0, i, 0)),
        pl.BlockSpec((2, _B, 128), lambda i: (0, i, 0)),
    ],
    out_specs=pl.BlockSpec((_B, F_OUT), lambda i: (i, 0)),
    out_shape=jax.ShapeDtypeStruct((NP, F_OUT), jnp.float32),
)


def kernel(x, edge_index, W1, b1, W2, b2):
    row = edge_index[0]
    col = edge_index[1]
    pad0 = jnp.zeros((TOT - E,), jnp.int32)
    padn = jnp.full((TOT - E,), NP, jnp.int32)
    row_g = jnp.concatenate([row, pad0]).reshape(TOT // K, K)
    row_s = jnp.concatenate([row, padn]).reshape(TOT // K, K)
    col_s = jnp.concatenate([col, padn]).reshape(TOT // K, K)
    ones128 = jnp.ones((K, 128), jnp.float32)
    zeros128 = jnp.zeros((SLAB, 128), jnp.float32)
    x_p = jnp.pad(x, ((0, NP - N), (0, 0)))

    W2Tp = jnp.pad(W2.T, ((0, 0), (0, F_HID - F_OUT)))
    b2p = jnp.pad(b2, (0, F_HID - F_OUT))[None, :]

    deg_parts = _deg_kernel(row_s, ones128, zeros128)
    h1 = _mm1(x_p, W1.T, b1[None, :], deg_parts)
    acc1 = _msg_hid(h1, row_g, col_s, zeros128)
    h2 = _post1(acc1, deg_parts, W2Tp, b2p)
    acc2 = _msg_out(h2, row_g, col_s, zeros128)
    return _post2(acc2, deg_parts)[:N]


# speedup vs baseline: 9.6822x; 1.0533x over previous
import functools

import jax
import jax.numpy as jnp
from jax import lax
from jax.experimental import pallas as pl
from jax.experimental.pallas import tpu as pltpu
from jax.experimental.pallas import tpu_sc as plsc

N = 10000
E = 320000
F_IN = 128
F_HID = 128
F_OUT = 64

K = 128
NW = 32
CH = 80
TOT = NW * K * CH
NP = 10240
NPAD = NP + 16
SLAB = NP // 16

_mesh = plsc.VectorSubcoreMesh(core_axis_name="c", subcore_axis_name="s")


def _deg_body(rows_hbm, ones_hbm, zeros_hbm, out_hbm, idx_s, ones_v, hist_sh):
    c = lax.axis_index("c")
    s = lax.axis_index("s")
    wid = c * 16 + s
    pltpu.sync_copy(rows_hbm.at[pl.ds(wid * CH, CH)], idx_s)
    pltpu.sync_copy(ones_hbm, ones_v)
    pltpu.sync_copy(zeros_hbm, hist_sh.at[pl.ds(s * SLAB, SLAB)])
    plsc.subcore_barrier()

    def step(j, carry):
        pltpu.sync_copy(ones_v, hist_sh.at[idx_s.at[j]], add=True)
        return carry

    lax.fori_loop(0, CH, step, 0)
    plsc.subcore_barrier()
    pltpu.sync_copy(hist_sh.at[pl.ds(s * SLAB, SLAB)],
                    out_hbm.at[c].at[pl.ds(s * SLAB, SLAB)])


_deg_kernel = pl.kernel(
    _deg_body,
    out_type=jax.ShapeDtypeStruct((2, NP, 128), jnp.float32),
    mesh=_mesh,
    scratch_types=[
        pltpu.VMEM((CH, K), jnp.int32),
        pltpu.VMEM((K, 128), jnp.float32),
        pltpu.VMEM_SHARED((NPAD, 128), jnp.float32),
    ],
)


_NBUF = 2
_CHP = 40
_CH_FAST = 120
_CH_SLOW = 40
_FAST_C = 0


def _msg_body(h_hbm, rowg_hbm, cols_hbm, z_hbm, out_hbm, idx_g, idx_s,
              b0, b1, s0, s1, acc_sh):
    bufs = (b0, b1)
    sems = (s0, s1)
    c = lax.axis_index("c")
    s = lax.axis_index("s")

    @pl.when(c == 0)
    def _():
        pltpu.sync_copy(h_hbm.at[pl.ds(s * SLAB, SLAB)],
                        acc_sh.at[pl.ds(s * SLAB, SLAB)])

    @pl.when(c == 1)
    def _():
        pltpu.sync_copy(z_hbm, acc_sh.at[pl.ds(s * SLAB, SLAB)])

    plsc.subcore_barrier()

    is_fast = c == _FAST_C
    tile_base = jnp.where(is_fast, s * _CH_FAST, 16 * _CH_FAST + s * _CH_SLOW)
    nph = jnp.where(is_fast, _CH_FAST // _CHP, _CH_SLOW // _CHP)

    for p in range(_CH_FAST // _CHP):
      @pl.when(p < nph)
      def _():
        base = tile_base + p * _CHP
        pltpu.sync_copy(rowg_hbm.at[pl.ds(base, _CHP)], idx_g)
        pltpu.sync_copy(cols_hbm.at[pl.ds(base, _CHP)], idx_s)
        for b in range(_NBUF):
            pltpu.async_copy(h_hbm.at[idx_g.at[b]], bufs[b], sems[b])

        def round_(t, carry):
            j = t * _NBUF
            for b in range(_NBUF):
                jj = j + b
                pltpu.make_async_copy(h_hbm.at[idx_g.at[jj]], bufs[b],
                                      sems[b]).wait()
                pltpu.sync_copy(bufs[b], acc_sh.at[idx_s.at[jj]], add=True)

                @pl.when(jj + _NBUF < _CHP)
                def _():
                    pltpu.async_copy(h_hbm.at[idx_g.at[jj + _NBUF]], bufs[b],
                                     sems[b])
            return carry

        lax.fori_loop(0, _CHP // _NBUF, round_, 0)

    plsc.subcore_barrier()
    pltpu.sync_copy(acc_sh.at[pl.ds(s * SLAB, SLAB)],
                    out_hbm.at[c].at[pl.ds(s * SLAB, SLAB)])


def _make_msg_kernel(F):
    return pl.kernel(
        _msg_body,
        out_type=jax.ShapeDtypeStruct((2, NP, F), jnp.float32),
        mesh=_mesh,
        scratch_types=(
            [pltpu.VMEM((_CHP, K), jnp.int32),
             pltpu.VMEM((_CHP, K), jnp.int32)]
            + [pltpu.VMEM((K, F), jnp.float32)] * _NBUF
            + [pltpu.SemaphoreType.DMA] * _NBUF
            + [pltpu.VMEM_SHARED((NPAD, F), jnp.float32)]
        ),
    )


_msg_hid = _make_msg_kernel(F_HID)
_msg_out = _make_msg_kernel(F_HID)

_B = 640


def _dis_of(dp_ref):
    deg = dp_ref[0, :, 0] + dp_ref[1, :, 0] + 1.0
    return lax.rsqrt(deg)


def _mm1_body(x_ref, w_ref, b_ref, dp_ref, o_ref):
    dis = _dis_of(dp_ref)
    h = jnp.dot(x_ref[...], w_ref[...], preferred_element_type=jnp.float32)
    o_ref[...] = (h + b_ref[...]) * dis[:, None]


_mm1 = pl.pallas_call(
    _mm1_body,
    grid=(NP // _B,),
    in_specs=[
        pl.BlockSpec((_B, F_IN), lambda i: (i, 0)),
        pl.BlockSpec((F_IN, F_HID), lambda i: (0, 0)),
        pl.BlockSpec((1, F_HID), lambda i: (0, 0)),
        pl.BlockSpec((2, _B, 128), lambda i: (0, i, 0)),
    ],
    out_specs=pl.BlockSpec((_B, F_HID), lambda i: (i, 0)),
    out_shape=jax.ShapeDtypeStruct((NP, F_HID), jnp.float32),
)


def _post1_body(a_ref, dp_ref, w_ref, b_ref, o_ref):
    dis = _dis_of(dp_ref)
    z = jnp.maximum((a_ref[0] + a_ref[1]) * dis[:, None], 0.0)
    h = jnp.dot(z, w_ref[...], preferred_element_type=jnp.float32)
    o_ref[...] = (h + b_ref[...]) * dis[:, None]


_post1 = pl.pallas_call(
    _post1_body,
    grid=(NP // _B,),
    in_specs=[
        pl.BlockSpec((2, _B, F_HID), lambda i: (0, i, 0)),
        pl.BlockSpec((2, _B, 128), lambda i: (0, i, 0)),
        pl.BlockSpec((F_HID, F_HID), lambda i: (0, 0)),
        pl.BlockSpec((1, F_HID), lambda i: (0, 0)),
    ],
    out_specs=pl.BlockSpec((_B, F_HID), lambda i: (i, 0)),
    out_shape=jax.ShapeDtypeStruct((NP, F_HID), jnp.float32),
)


def _post2_body(a_ref, dp_ref, o_ref):
    dis = _dis_of(dp_ref)
    z = (a_ref[0, :, :F_OUT] + a_ref[1, :, :F_OUT]) * dis[:, None]
    m = jnp.max(z, axis=1, keepdims=True)
    e = jnp.exp(z - m)
    o_ref[...] = (z - m) - jnp.log(jnp.sum(e, axis=1, keepdims=True))


_post2 = pl.pallas_call(
    _post2_body,
    grid=(NP // _B,),
    in_specs=[
        pl.BlockSpec((2, _B, F_HID), lambda i: (0, i, 0)),
        pl.BlockSpec((2, _B, 128), lambda i: (0, i, 0)),
    ],
    out_specs=pl.BlockSpec((_B, F_OUT), lambda i: (i, 0)),
    out_shape=jax.ShapeDtypeStruct((NP, F_OUT), jnp.float32),
)


def kernel(x, edge_index, W1, b1, W2, b2):
    row = edge_index[0]
    col = edge_index[1]
    pad0 = jnp.zeros((TOT - E,), jnp.int32)
    padn = jnp.full((TOT - E,), NP, jnp.int32)
    row_g = jnp.concatenate([row, pad0]).reshape(TOT // K, K)
    row_s = jnp.concatenate([row, padn]).reshape(TOT // K, K)
    col_s = jnp.concatenate([col, padn]).reshape(TOT // K, K)
    ones128 = jnp.ones((K, 128), jnp.float32)
    zeros128 = jnp.zeros((SLAB, 128), jnp.float32)
    x_p = jnp.pad(x, ((0, NP - N), (0, 0)))

    W2Tp = jnp.pad(W2.T, ((0, 0), (0, F_HID - F_OUT)))
    b2p = jnp.pad(b2, (0, F_HID - F_OUT))[None, :]

    deg_parts = _deg_kernel(row_s, ones128, zeros128)
    h1 = _mm1(x_p, W1.T, b1[None, :], deg_parts)
    acc1 = _msg_hid(h1, row_g, col_s, zeros128)
    h2 = _post1(acc1, deg_parts, W2Tp, b2p)
    acc2 = _msg_out(h2, row_g, col_s, zeros128)
    return _post2(acc2, deg_parts)[:N]


# speedup vs baseline: 26.7624x; 2.7641x over previous
import functools

import jax
import jax.numpy as jnp
from jax import lax
from jax.experimental import pallas as pl
from jax.experimental.pallas import tpu as pltpu
from jax.experimental.pallas import tpu_sc as plsc

N = 10000
E = 320000
F_IN = 128
F_HID = 128
F_OUT = 64

K = 128
NW = 32
CH = 80
TOT = NW * K * CH
NP = 10240
NPAD = NP + 16
SLAB = NP // 16

_mesh = plsc.VectorSubcoreMesh(core_axis_name="c", subcore_axis_name="s")


def _deg_body(rows_hbm, ones_hbm, zeros_hbm, out_hbm, idx_s, ones_v, hist_sh):
    c = lax.axis_index("c")
    s = lax.axis_index("s")
    wid = c * 16 + s
    pltpu.sync_copy(rows_hbm.at[pl.ds(wid * CH, CH)], idx_s)
    pltpu.sync_copy(ones_hbm, ones_v)
    pltpu.sync_copy(zeros_hbm, hist_sh.at[pl.ds(s * SLAB, SLAB)])
    plsc.subcore_barrier()

    def step(j, carry):
        pltpu.sync_copy(ones_v, hist_sh.at[idx_s.at[j]], add=True)
        return carry

    lax.fori_loop(0, CH, step, 0)
    plsc.subcore_barrier()
    pltpu.sync_copy(hist_sh.at[pl.ds(s * SLAB, SLAB)],
                    out_hbm.at[c].at[pl.ds(s * SLAB, SLAB)])


_deg_kernel = pl.kernel(
    _deg_body,
    out_type=jax.ShapeDtypeStruct((2, NP, 128), jnp.float32),
    mesh=_mesh,
    scratch_types=[
        pltpu.VMEM((CH, K), jnp.int32),
        pltpu.VMEM((K, 128), jnp.float32),
        pltpu.VMEM_SHARED((NPAD, 128), jnp.float32),
    ],
)


_NBUF = 2
_CHP = 40
_CH_FAST = 80
_CH_SLOW = 80
_FAST_C = 0


def _msg_body(h_hbm, rowg_hbm, cols_hbm, z_hbm, out_hbm, idx_g, idx_s,
              b0, b1, s0, s1, acc_sh):
    bufs = (b0, b1)
    sems = (s0, s1)
    c = lax.axis_index("c")
    s = lax.axis_index("s")

    @pl.when(c == 0)
    def _():
        pltpu.sync_copy(h_hbm.at[pl.ds(s * SLAB, SLAB)],
                        acc_sh.at[pl.ds(s * SLAB, SLAB)])

    @pl.when(c == 1)
    def _():
        pltpu.sync_copy(z_hbm, acc_sh.at[pl.ds(s * SLAB, SLAB)])

    plsc.subcore_barrier()

    is_fast = c == _FAST_C
    tile_base = jnp.where(is_fast, s * _CH_FAST, 16 * _CH_FAST + s * _CH_SLOW)
    nph = jnp.where(is_fast, _CH_FAST // _CHP, _CH_SLOW // _CHP)

    for p in range(_CH_FAST // _CHP):
      @pl.when(p < nph)
      def _():
        base = tile_base + p * _CHP
        pltpu.sync_copy(rowg_hbm.at[pl.ds(base, _CHP)], idx_g)
        pltpu.sync_copy(cols_hbm.at[pl.ds(base, _CHP)], idx_s)
        for b in range(_NBUF):
            pltpu.async_copy(h_hbm.at[idx_g.at[b]], bufs[b], sems[b])

        def round_(t, carry):
            j = t * _NBUF
            for b in range(_NBUF):
                jj = j + b
                pltpu.make_async_copy(h_hbm.at[idx_g.at[jj]], bufs[b],
                                      sems[b]).wait()
                pltpu.sync_copy(bufs[b], acc_sh.at[idx_s.at[jj]], add=True)

                @pl.when(jj + _NBUF < _CHP)
                def _():
                    pltpu.async_copy(h_hbm.at[idx_g.at[jj + _NBUF]], bufs[b],
                                     sems[b])
            return carry

        lax.fori_loop(0, _CHP // _NBUF, round_, 0)

    plsc.subcore_barrier()
    pltpu.sync_copy(acc_sh.at[pl.ds(s * SLAB, SLAB)],
                    out_hbm.at[c].at[pl.ds(s * SLAB, SLAB)])


def _make_msg_kernel(F):
    return pl.kernel(
        _msg_body,
        out_type=jax.ShapeDtypeStruct((2, NP, F), jnp.float32),
        mesh=_mesh,
        scratch_types=(
            [pltpu.VMEM((_CHP, K), jnp.int32),
             pltpu.VMEM((_CHP, K), jnp.int32)]
            + [pltpu.VMEM((K, F), jnp.float32)] * _NBUF
            + [pltpu.SemaphoreType.DMA] * _NBUF
            + [pltpu.VMEM_SHARED((NPAD, F), jnp.float32)]
        ),
    )


_msg_hid = _make_msg_kernel(F_HID)
_msg_out = _make_msg_kernel(F_HID)

_B = 640


def _dis_of(dp_ref):
    deg = dp_ref[0, :, 0] + dp_ref[1, :, 0] + 1.0
    return lax.rsqrt(deg)


def _mm1_body(x_ref, w_ref, b_ref, dp_ref, o_ref):
    dis = _dis_of(dp_ref)
    h = jnp.dot(x_ref[...], w_ref[...], preferred_element_type=jnp.float32)
    o_ref[...] = (h + b_ref[...]) * dis[:, None]


_mm1 = pl.pallas_call(
    _mm1_body,
    grid=(NP // _B,),
    in_specs=[
        pl.BlockSpec((_B, F_IN), lambda i: (i, 0)),
        pl.BlockSpec((F_IN, F_HID), lambda i: (0, 0)),
        pl.BlockSpec((1, F_HID), lambda i: (0, 0)),
        pl.BlockSpec((2, _B, 128), lambda i: (0, i, 0)),
    ],
    out_specs=pl.BlockSpec((_B, F_HID), lambda i: (i, 0)),
    out_shape=jax.ShapeDtypeStruct((NP, F_HID), jnp.float32),
)


def _post1_body(a_ref, dp_ref, w_ref, b_ref, o_ref):
    dis = _dis_of(dp_ref)
    z = jnp.maximum((a_ref[0] + a_ref[1]) * dis[:, None], 0.0)
    h = jnp.dot(z, w_ref[...], preferred_element_type=jnp.float32)
    o_ref[...] = (h + b_ref[...]) * dis[:, None]


_post1 = pl.pallas_call(
    _post1_body,
    grid=(NP // _B,),
    in_specs=[
        pl.BlockSpec((2, _B, F_HID), lambda i: (0, i, 0)),
        pl.BlockSpec((2, _B, 128), lambda i: (0, i, 0)),
        pl.BlockSpec((F_HID, F_HID), lambda i: (0, 0)),
        pl.BlockSpec((1, F_HID), lambda i: (0, 0)),
    ],
    out_specs=pl.BlockSpec((_B, F_HID), lambda i: (i, 0)),
    out_shape=jax.ShapeDtypeStruct((NP, F_HID), jnp.float32),
)


def _post2_body(a_ref, dp_ref, o_ref):
    dis = _dis_of(dp_ref)
    z = (a_ref[0, :, :F_OUT] + a_ref[1, :, :F_OUT]) * dis[:, None]
    m = jnp.max(z, axis=1, keepdims=True)
    e = jnp.exp(z - m)
    o_ref[...] = (z - m) - jnp.log(jnp.sum(e, axis=1, keepdims=True))


_post2 = pl.pallas_call(
    _post2_body,
    grid=(NP // _B,),
    in_specs=[
        pl.BlockSpec((2, _B, F_HID), lambda i: (0, i, 0)),
        pl.BlockSpec((2, _B, 128), lambda i: (0, i, 0)),
    ],
    out_specs=pl.BlockSpec((_B, F_OUT), lambda i: (i, 0)),
    out_shape=jax.ShapeDtypeStruct((NP, F_OUT), jnp.float32),
)


def kernel(x, edge_index, W1, b1, W2, b2):
    row = edge_index[0]
    col = edge_index[1]
    ar = jnp.arange(TOT - E, dtype=jnp.int32)
    pad0 = ar % N
    padn = NP + (ar % 16)
    row_g = jnp.concatenate([row, pad0]).reshape(TOT // K, K)
    row_s = jnp.concatenate([row, padn]).reshape(TOT // K, K)
    col_s = jnp.concatenate([col, padn]).reshape(TOT // K, K)
    ones128 = jnp.ones((K, 128), jnp.float32)
    zeros128 = jnp.zeros((SLAB, 128), jnp.float32)
    x_p = jnp.pad(x, ((0, NP - N), (0, 0)))

    W2Tp = jnp.pad(W2.T, ((0, 0), (0, F_HID - F_OUT)))
    b2p = jnp.pad(b2, (0, F_HID - F_OUT))[None, :]

    deg_parts = _deg_kernel(row_s, ones128, zeros128)
    h1 = _mm1(x_p, W1.T, b1[None, :], deg_parts)
    acc1 = _msg_hid(h1, row_g, col_s, zeros128)
    h2 = _post1(acc1, deg_parts, W2Tp, b2p)
    acc2 = _msg_out(h2, row_g, col_s, zeros128)
    return _post2(acc2, deg_parts)[:N]


# speedup vs baseline: 26.8571x; 1.0035x over previous
import functools

import jax
import jax.numpy as jnp
from jax import lax
from jax.experimental import pallas as pl
from jax.experimental.pallas import tpu as pltpu
from jax.experimental.pallas import tpu_sc as plsc

N = 10000
E = 320000
F_IN = 128
F_HID = 128
F_OUT = 64

K = 128
NW = 32
CH = 80
TOT = NW * K * CH
NP = 10240
NPAD = NP + 16
SLAB = NP // 16

_mesh = plsc.VectorSubcoreMesh(core_axis_name="c", subcore_axis_name="s")


def _deg_body(rows_hbm, ones_hbm, zeros_hbm, out_hbm, idx_s, ones_v, hist_sh):
    c = lax.axis_index("c")
    s = lax.axis_index("s")
    wid = c * 16 + s
    pltpu.sync_copy(rows_hbm.at[pl.ds(wid * CH, CH)], idx_s)
    pltpu.sync_copy(ones_hbm, ones_v)
    pltpu.sync_copy(zeros_hbm, hist_sh.at[pl.ds(s * SLAB, SLAB)])
    plsc.subcore_barrier()

    def step(j, carry):
        pltpu.sync_copy(ones_v, hist_sh.at[idx_s.at[j]], add=True)
        return carry

    lax.fori_loop(0, CH, step, 0)
    plsc.subcore_barrier()
    pltpu.sync_copy(hist_sh.at[pl.ds(s * SLAB, SLAB)],
                    out_hbm.at[c].at[pl.ds(s * SLAB, SLAB)])


_deg_kernel = pl.kernel(
    _deg_body,
    out_type=jax.ShapeDtypeStruct((2, NP, 128), jnp.float32),
    mesh=_mesh,
    scratch_types=[
        pltpu.VMEM((CH, K), jnp.int32),
        pltpu.VMEM((K, 128), jnp.float32),
        pltpu.VMEM_SHARED((NP, 128), jnp.float32),
    ],
)


_NBUF = 2
_CHP = 40
_CH_FAST = 80
_CH_SLOW = 80
_FAST_C = 0


def _msg_body(h_hbm, rowg_hbm, cols_hbm, z_hbm, out_hbm, idx_g, idx_s,
              b0, b1, s0, s1, acc_sh):
    bufs = (b0, b1)
    sems = (s0, s1)
    c = lax.axis_index("c")
    s = lax.axis_index("s")

    @pl.when(c == 0)
    def _():
        pltpu.sync_copy(h_hbm.at[pl.ds(s * SLAB, SLAB)],
                        acc_sh.at[pl.ds(s * SLAB, SLAB)])

    @pl.when(c == 1)
    def _():
        pltpu.sync_copy(z_hbm, acc_sh.at[pl.ds(s * SLAB, SLAB)])

    plsc.subcore_barrier()

    is_fast = c == _FAST_C
    tile_base = jnp.where(is_fast, s * _CH_FAST, 16 * _CH_FAST + s * _CH_SLOW)
    nph = jnp.where(is_fast, _CH_FAST // _CHP, _CH_SLOW // _CHP)

    for p in range(_CH_FAST // _CHP):
      @pl.when(p < nph)
      def _():
        base = tile_base + p * _CHP
        pltpu.sync_copy(rowg_hbm.at[pl.ds(base, _CHP)], idx_g)
        pltpu.sync_copy(cols_hbm.at[pl.ds(base, _CHP)], idx_s)
        for b in range(_NBUF):
            pltpu.async_copy(h_hbm.at[idx_g.at[b]], bufs[b], sems[b])

        def round_(t, carry):
            j = t * _NBUF
            for b in range(_NBUF):
                jj = j + b
                pltpu.make_async_copy(h_hbm.at[idx_g.at[jj]], bufs[b],
                                      sems[b]).wait()
                pltpu.sync_copy(bufs[b], acc_sh.at[idx_s.at[jj]], add=True)

                @pl.when(jj + _NBUF < _CHP)
                def _():
                    pltpu.async_copy(h_hbm.at[idx_g.at[jj + _NBUF]], bufs[b],
                                     sems[b])
            return carry

        lax.fori_loop(0, _CHP // _NBUF, round_, 0)

    plsc.subcore_barrier()
    pltpu.sync_copy(acc_sh.at[pl.ds(s * SLAB, SLAB)],
                    out_hbm.at[c].at[pl.ds(s * SLAB, SLAB)])


def _make_msg_kernel(F):
    return pl.kernel(
        _msg_body,
        out_type=jax.ShapeDtypeStruct((2, NP, F), jnp.float32),
        mesh=_mesh,
        scratch_types=(
            [pltpu.VMEM((_CHP, K), jnp.int32),
             pltpu.VMEM((_CHP, K), jnp.int32)]
            + [pltpu.VMEM((K, F), jnp.float32)] * _NBUF
            + [pltpu.SemaphoreType.DMA] * _NBUF
            + [pltpu.VMEM_SHARED((NP, F), jnp.float32)]
        ),
    )


_msg_hid = _make_msg_kernel(F_HID)
_msg_out = _make_msg_kernel(F_HID)

_B = 640


def _dis_of(dp_ref):
    deg = dp_ref[0, :, 0] + dp_ref[1, :, 0] + 1.0
    return lax.rsqrt(deg)


def _row_mask(i):
    rows = i * _B + lax.broadcasted_iota(jnp.int32, (_B, 1), 0)
    return rows < N


def _mm1_body(x_ref, w_ref, b_ref, dp_ref, o_ref, dis_ref):
    dis = _dis_of(dp_ref)
    h = jnp.dot(x_ref[...], w_ref[...], preferred_element_type=jnp.float32)
    o_ref[...] = jnp.where(_row_mask(pl.program_id(0)),
                           (h + b_ref[...]) * dis[:, None], 0.0)
    dis_ref[...] = jnp.broadcast_to(dis[:, None], (_B, 16))


_mm1 = pl.pallas_call(
    _mm1_body,
    grid=(NP // _B,),
    in_specs=[
        pl.BlockSpec((_B, F_IN), lambda i: (i, 0)),
        pl.BlockSpec((F_IN, F_HID), lambda i: (0, 0)),
        pl.BlockSpec((1, F_HID), lambda i: (0, 0)),
        pl.BlockSpec((2, _B, 128), lambda i: (0, i, 0)),
    ],
    out_specs=[pl.BlockSpec((_B, F_HID), lambda i: (i, 0)),
               pl.BlockSpec((_B, 16), lambda i: (i, 0))],
    out_shape=[jax.ShapeDtypeStruct((NP, F_HID), jnp.float32),
               jax.ShapeDtypeStruct((NP, 16), jnp.float32)],
)


def _post1_body(a_ref, dp_ref, w_ref, b_ref, o_ref):
    dis = dp_ref[:, 0]
    z = jnp.maximum((a_ref[0] + a_ref[1]) * dis[:, None], 0.0)
    h = jnp.dot(z, w_ref[...], preferred_element_type=jnp.float32)
    o_ref[...] = jnp.where(_row_mask(pl.program_id(0)),
                           (h + b_ref[...]) * dis[:, None], 0.0)


_post1 = pl.pallas_call(
    _post1_body,
    grid=(NP // _B,),
    in_specs=[
        pl.BlockSpec((2, _B, F_HID), lambda i: (0, i, 0)),
        pl.BlockSpec((_B, 16), lambda i: (i, 0)),
        pl.BlockSpec((F_HID, F_HID), lambda i: (0, 0)),
        pl.BlockSpec((1, F_HID), lambda i: (0, 0)),
    ],
    out_specs=pl.BlockSpec((_B, F_HID), lambda i: (i, 0)),
    out_shape=jax.ShapeDtypeStruct((NP, F_HID), jnp.float32),
)


def _post2_body(a_ref, dp_ref, o_ref):
    dis = dp_ref[:, 0]
    z = (a_ref[0, :, :F_OUT] + a_ref[1, :, :F_OUT]) * dis[:, None]
    m = jnp.max(z, axis=1, keepdims=True)
    e = jnp.exp(z - m)
    o_ref[...] = (z - m) - jnp.log(jnp.sum(e, axis=1, keepdims=True))


_post2 = pl.pallas_call(
    _post2_body,
    grid=(NP // _B,),
    in_specs=[
        pl.BlockSpec((2, _B, F_HID), lambda i: (0, i, 0)),
        pl.BlockSpec((_B, 16), lambda i: (i, 0)),
    ],
    out_specs=pl.BlockSpec((_B, F_OUT), lambda i: (i, 0)),
    out_shape=jax.ShapeDtypeStruct((NP, F_OUT), jnp.float32),
)


def kernel(x, edge_index, W1, b1, W2, b2):
    row = edge_index[0]
    col = edge_index[1]
    ar = jnp.arange(TOT - E, dtype=jnp.int32)
    padz = N + (ar % (NP - N))
    padr = ar % N
    row_g = jnp.concatenate([row, padz]).reshape(TOT // K, K)
    col_s = jnp.concatenate([col, padr]).reshape(TOT // K, K)
    ones128 = jnp.ones((K, 128), jnp.float32)
    zeros128 = jnp.zeros((SLAB, 128), jnp.float32)
    x_p = jnp.pad(x, ((0, NP - N), (0, 0)))

    W2Tp = jnp.pad(W2.T, ((0, 0), (0, F_HID - F_OUT)))
    b2p = jnp.pad(b2, (0, F_HID - F_OUT))[None, :]

    deg_parts = _deg_kernel(row_g, ones128, zeros128)
    h1, dis16 = _mm1(x_p, W1.T, b1[None, :], deg_parts)
    acc1 = _msg_hid(h1, row_g, col_s, zeros128)
    h2 = _post1(acc1, dis16, W2Tp, b2p)
    acc2 = _msg_out(h2, row_g, col_s, zeros128)
    return _post2(acc2, dis16)[:N]
